# baseline (speedup 1.0000x reference)
import functools

import jax
import jax.numpy as jnp
from jax import lax
from jax.experimental import pallas as pl
from jax.experimental.pallas import tpu as pltpu
from jax.experimental.pallas import tpu_sc as plsc

BINS = 30
ROWS, COLS = 16384, 256
TOT = ROWS * COLS

NC, NS, LANES = 2, 16, 16
NW = NC * NS

QSHIFT = 15
QSCALE = float(1 << QSHIFT)
QMAX = (1 << 19) - 1

BLK_A = 2048
GRID_A = ROWS // BLK_A

NSPLIT = 2
ROWS_H = ROWS // NSPLIT
ROWS_W = ROWS_H // NW
ROWS_CH = 128
NCH = ROWS_W // ROWS_CH
COL_GROUPS = COLS // LANES
NSLOT = 32


def _pack_kernel(x_ref, t_ref, out_ref):
    x = x_ref[...]
    t = t_ref[...]
    u = jnp.exp(-jnp.abs(x))
    one_u = 1.0 + u
    r = 1.0 / one_u
    sig = jnp.where(x >= 0.0, r, 1.0 - r)
    g30 = jnp.abs(sig - t) * float(BINS)
    b = jnp.minimum(jnp.floor(g30), float(BINS - 1)).astype(jnp.int32)
    e = jnp.maximum(x, 0.0) - x * t + jnp.log(one_u)
    q = jnp.minimum((e * QSCALE).astype(jnp.int32), QMAX)
    out_ref[...] = jnp.bitwise_or(jnp.left_shift(q, 9), jnp.left_shift(b, 4))


NACC = 4


def _sc_hist(packed_hbm, out_hbm, buf0, buf1,
             s0, s1, s2, s3, c0, c1, c2, c3, fs, fc, sem0, sem1):
    wid = lax.axis_index("c") * NS + lax.axis_index("s")
    row0 = wid * ROWS_W
    accs = (s0, s1, s2, s3)
    accc = (c0, c1, c2, c3)

    zeros16 = jnp.zeros((LANES,), jnp.int32)
    for j in range(NSLOT):
        for a in range(NACC):
            accs[a][pl.ds(j * LANES, LANES)] = zeros16
            accc[a][pl.ds(j * LANES, LANES)] = zeros16

    lane = lax.iota(jnp.int32, LANES)
    ones16 = jnp.ones((LANES,), jnp.int32)
    bufs = (buf0, buf1)
    sems = (sem0, sem1)

    def start(c):
        return pltpu.async_copy(
            packed_hbm.at[pl.ds(row0 + c * ROWS_CH, ROWS_CH), :],
            bufs[c % 2], sems[c % 2])

    def process(buf):
        def row_body(r, _):
            idxs, qs = [], []
            for k in range(COL_GROUPS):
                w = buf[r, pl.ds(k * LANES, LANES)]
                idxs.append(jnp.bitwise_and(w, 0x1F0) + lane)
                qs.append(lax.shift_right_logical(w, 9))
            for k in range(COL_GROUPS):
                plsc.addupdate_scatter(accs[k % NACC], [idxs[k]], qs[k])
                plsc.addupdate_scatter(accc[k % NACC], [idxs[k]], ones16)
            return _

        lax.fori_loop(0, ROWS_CH, row_body, None)

    descs = [start(0)]
    for c in range(NCH):
        if c + 1 < NCH:
            descs.append(start(c + 1))
        descs[c].wait()
        process(bufs[c % 2])

    for j in range(NSLOT):
        sl = pl.ds(j * LANES, LANES)
        ssum = (s0[sl] + s1[sl]) + (s2[sl] + s3[sl])
        csum = (c0[sl] + c1[sl]) + (c2[sl] + c3[sl])
        fs[sl] = ssum.astype(jnp.float32) * (1.0 / QSCALE)
        fc[sl] = csum.astype(jnp.float32)

    pltpu.sync_copy(fs, out_hbm.at[0, wid])
    pltpu.sync_copy(fc, out_hbm.at[1, wid])


def _finish_kernel(p0_ref, p1_ref, out_ref):
    s = jnp.sum(p0_ref[0], axis=(0, 2)) + jnp.sum(p1_ref[0], axis=(0, 2))
    c = jnp.sum(p0_ref[1], axis=(0, 2)) + jnp.sum(p1_ref[1], axis=(0, 2))
    nz = c > 0.0
    n = jnp.sum(jnp.where(nz, 1.0, 0.0))
    total = jnp.sum(jnp.where(nz, s / jnp.maximum(c, 1.0), 0.0))
    out_ref[0, 0] = total / jnp.maximum(n, 1.0)


def kernel(input, target):
    grid_h = ROWS_H // BLK_A

    def pack_half(h):
        return pl.pallas_call(
            _pack_kernel,
            grid=(grid_h,),
            in_specs=[
                pl.BlockSpec((BLK_A, COLS), lambda i, h=h: (i + h * grid_h, 0)),
                pl.BlockSpec((BLK_A, COLS), lambda i, h=h: (i + h * grid_h, 0)),
            ],
            out_specs=pl.BlockSpec((BLK_A, COLS), lambda i: (i, 0)),
            out_shape=jax.ShapeDtypeStruct((ROWS_H, COLS), jnp.int32),
            compiler_params=pltpu.CompilerParams(
                dimension_semantics=("arbitrary",),
            ),
        )(input, target)

    sc_hist = functools.partial(
        pl.kernel,
        mesh=plsc.VectorSubcoreMesh(core_axis_name="c", subcore_axis_name="s"),
        out_type=jax.ShapeDtypeStruct((2, NW, NSLOT * LANES), jnp.float32),
        scratch_types=[
            pltpu.VMEM((ROWS_CH, COLS), jnp.int32),
            pltpu.VMEM((ROWS_CH, COLS), jnp.int32),
        ] + [pltpu.VMEM((NSLOT * LANES,), jnp.int32) for _ in range(2 * NACC)] + [
            pltpu.VMEM((NSLOT * LANES,), jnp.float32),
            pltpu.VMEM((NSLOT * LANES,), jnp.float32),
            pltpu.SemaphoreType.DMA,
            pltpu.SemaphoreType.DMA,
        ],
        compiler_params=pltpu.CompilerParams(needs_layout_passes=False),
    )(_sc_hist)

    p0 = sc_hist(pack_half(0)).reshape(2, NW, NSLOT, LANES)
    p1 = sc_hist(pack_half(1)).reshape(2, NW, NSLOT, LANES)

    out = pl.pallas_call(
        _finish_kernel,
        out_specs=pl.BlockSpec(memory_space=pltpu.SMEM),
        out_shape=jax.ShapeDtypeStruct((1, 1), jnp.float32),
    )(p0, p1)
    return out[0, 0]


# speedup vs baseline: 1.0267x; 1.0267x over previous
import functools

import jax
import jax.numpy as jnp
from jax import lax
from jax.experimental import pallas as pl
from jax.experimental.pallas import tpu as pltpu
from jax.experimental.pallas import tpu_sc as plsc

BINS = 30
ROWS, COLS = 16384, 256
TOT = ROWS * COLS

NC, NS, LANES = 2, 16, 16
NW = NC * NS

QSHIFT = 15
QSCALE = float(1 << QSHIFT)
QMAX = (1 << 19) - 1

BLK_A = 2048
GRID_A = ROWS // BLK_A

ROWS_W = ROWS // NW
ROWS_CH = 128
NCH = ROWS_W // ROWS_CH
COL_GROUPS = COLS // LANES
NSLOT = 32


def _pack_kernel(x_ref, t_ref, out_ref):
    x = x_ref[...]
    t = t_ref[...]
    u = jnp.exp(-jnp.abs(x))
    one_u = 1.0 + u
    r = 1.0 / one_u
    sig = jnp.where(x >= 0.0, r, 1.0 - r)
    g30 = jnp.abs(sig - t) * float(BINS)
    b = jnp.minimum(jnp.floor(g30), float(BINS - 1)).astype(jnp.int32)
    e = jnp.maximum(x, 0.0) - x * t + jnp.log(one_u)
    q = jnp.minimum((e * QSCALE).astype(jnp.int32), QMAX)
    out_ref[...] = jnp.bitwise_or(jnp.left_shift(q, 9), jnp.left_shift(b, 4))


NACC = 4


def _sc_hist(packed_hbm, out_hbm, buf0, buf1,
             s0, s1, s2, s3, c0, c1, c2, c3, fs, fc, sem0, sem1):
    wid = lax.axis_index("c") * NS + lax.axis_index("s")
    row0 = wid * ROWS_W
    accs = (s0, s1, s2, s3)
    accc = (c0, c1, c2, c3)

    zeros16 = jnp.zeros((LANES,), jnp.int32)
    for j in range(NSLOT):
        for a in range(NACC):
            accs[a][pl.ds(j * LANES, LANES)] = zeros16
            accc[a][pl.ds(j * LANES, LANES)] = zeros16

    lane = lax.iota(jnp.int32, LANES)
    ones16 = jnp.ones((LANES,), jnp.int32)
    bufs = (buf0, buf1)
    sems = (sem0, sem1)

    def start(c):
        return pltpu.async_copy(
            packed_hbm.at[pl.ds(row0 + c * ROWS_CH, ROWS_CH), :],
            bufs[c % 2], sems[c % 2])

    def process(buf):
        @plsc.parallel_loop(0, ROWS_CH)
        def row_body(r):
            idxs, qs = [], []
            for k in range(COL_GROUPS):
                w = buf[r, pl.ds(k * LANES, LANES)]
                idxs.append(jnp.bitwise_and(w, 0x1F0) + lane)
                qs.append(lax.shift_right_logical(w, 9))
            for k in range(COL_GROUPS):
                plsc.addupdate_scatter(accs[k % NACC], [idxs[k]], qs[k])
                plsc.addupdate_scatter(accc[k % NACC], [idxs[k]], ones16)

    descs = [start(0)]
    for c in range(NCH):
        if c + 1 < NCH:
            descs.append(start(c + 1))
        descs[c].wait()
        process(bufs[c % 2])

    for j in range(NSLOT):
        sl = pl.ds(j * LANES, LANES)
        ssum = (s0[sl] + s1[sl]) + (s2[sl] + s3[sl])
        csum = (c0[sl] + c1[sl]) + (c2[sl] + c3[sl])
        fs[sl] = ssum.astype(jnp.float32) * (1.0 / QSCALE)
        fc[sl] = csum.astype(jnp.float32)

    pltpu.sync_copy(fs, out_hbm.at[0, wid])
    pltpu.sync_copy(fc, out_hbm.at[1, wid])


def _finish_kernel(p_ref, out_ref):
    s = jnp.sum(p_ref[0], axis=(0, 2))
    c = jnp.sum(p_ref[1], axis=(0, 2))
    nz = c > 0.0
    n = jnp.sum(jnp.where(nz, 1.0, 0.0))
    total = jnp.sum(jnp.where(nz, s / jnp.maximum(c, 1.0), 0.0))
    out_ref[0, 0] = total / jnp.maximum(n, 1.0)


def kernel(input, target):
    packed = pl.pallas_call(
        _pack_kernel,
        grid=(GRID_A,),
        in_specs=[
            pl.BlockSpec((BLK_A, COLS), lambda i: (i, 0)),
            pl.BlockSpec((BLK_A, COLS), lambda i: (i, 0)),
        ],
        out_specs=pl.BlockSpec((BLK_A, COLS), lambda i: (i, 0)),
        out_shape=jax.ShapeDtypeStruct((ROWS, COLS), jnp.int32),
        compiler_params=pltpu.CompilerParams(
            dimension_semantics=("arbitrary",),
        ),
    )(input, target)

    sc_hist = functools.partial(
        pl.kernel,
        mesh=plsc.VectorSubcoreMesh(core_axis_name="c", subcore_axis_name="s"),
        out_type=jax.ShapeDtypeStruct((2, NW, NSLOT * LANES), jnp.float32),
        scratch_types=[
            pltpu.VMEM((ROWS_CH, COLS), jnp.int32),
            pltpu.VMEM((ROWS_CH, COLS), jnp.int32),
        ] + [pltpu.VMEM((NSLOT * LANES,), jnp.int32) for _ in range(2 * NACC)] + [
            pltpu.VMEM((NSLOT * LANES,), jnp.float32),
            pltpu.VMEM((NSLOT * LANES,), jnp.float32),
            pltpu.SemaphoreType.DMA,
            pltpu.SemaphoreType.DMA,
        ],
        compiler_params=pltpu.CompilerParams(needs_layout_passes=False),
    )(_sc_hist)
    partials = sc_hist(packed)

    partials = partials.reshape(2, NW, NSLOT, LANES)

    out = pl.pallas_call(
        _finish_kernel,
        out_specs=pl.BlockSpec(memory_space=pltpu.SMEM),
        out_shape=jax.ShapeDtypeStruct((1, 1), jnp.float32),
    )(partials)
    return out[0, 0]


# speedup vs baseline: 1.0273x; 1.0007x over previous
import functools

import jax
import jax.numpy as jnp
from jax import lax
from jax.experimental import pallas as pl
from jax.experimental.pallas import tpu as pltpu
from jax.experimental.pallas import tpu_sc as plsc

BINS = 30
ROWS, COLS = 16384, 256
TOT = ROWS * COLS

NC, NS, LANES = 2, 16, 16
NW = NC * NS

QSHIFT = 15
QSCALE = float(1 << QSHIFT)
QMAX = (1 << 19) - 1

BLK_A = 2048
GRID_A = ROWS // BLK_A

ROWS_W = ROWS // NW
ROWS_CH = 128
NCH = ROWS_W // ROWS_CH
COL_GROUPS = COLS // LANES
NSLOT = 32


def _pack_kernel(x_ref, t_ref, out_ref):
    x = x_ref[...]
    t = t_ref[...]
    u = jnp.exp(-jnp.abs(x))
    one_u = 1.0 + u
    r = 1.0 / one_u
    sig = jnp.where(x >= 0.0, r, 1.0 - r)
    g30 = jnp.abs(sig - t) * float(BINS)
    b = jnp.minimum(jnp.floor(g30), float(BINS - 1)).astype(jnp.int32)
    e = jnp.maximum(x, 0.0) - x * t + jnp.log(one_u)
    q = jnp.minimum((e * QSCALE).astype(jnp.int32), QMAX)
    out_ref[...] = jnp.bitwise_or(jnp.left_shift(q, 9), jnp.left_shift(b, 4))


NACC = 4


def _sc_hist(packed_hbm, out_hbm, buf0, buf1,
             s0, s1, s2, s3, c0, c1, c2, c3, fs, fc, sem0, sem1):
    wid = lax.axis_index("c") * NS + lax.axis_index("s")
    row0 = wid * ROWS_W
    accs = (s0, s1, s2, s3)
    accc = (c0, c1, c2, c3)

    zeros16 = jnp.zeros((LANES,), jnp.int32)
    for j in range(NSLOT):
        for a in range(NACC):
            accs[a][pl.ds(j * LANES, LANES)] = zeros16
            accc[a][pl.ds(j * LANES, LANES)] = zeros16

    lane = lax.iota(jnp.int32, LANES)
    ones16 = jnp.ones((LANES,), jnp.int32)
    bufs = (buf0, buf1)
    sems = (sem0, sem1)

    def start(c):
        return pltpu.async_copy(
            packed_hbm.at[pl.ds(row0 + c * ROWS_CH, ROWS_CH), :],
            bufs[c % 2], sems[c % 2])

    def process(buf):
        @plsc.parallel_loop(0, ROWS_CH)
        def row_body(r):
            idxs, qs = [], []
            for k in range(COL_GROUPS):
                w = buf[r, pl.ds(k * LANES, LANES)]
                idxs.append(jnp.bitwise_and(w, 0x1F0) + lane)
                qs.append(lax.shift_right_logical(w, 9))
            for k in range(COL_GROUPS):
                plsc.addupdate_scatter(accs[k % NACC], [idxs[k]], qs[k])
                plsc.addupdate_scatter(accc[k % NACC], [idxs[k]], ones16)

    descs = [start(0)]
    for c in range(NCH):
        if c + 1 < NCH:
            descs.append(start(c + 1))
        descs[c].wait()
        process(bufs[c % 2])

    for j in range(NSLOT):
        sl = pl.ds(j * LANES, LANES)
        ssum = (s0[sl] + s1[sl]) + (s2[sl] + s3[sl])
        csum = (c0[sl] + c1[sl]) + (c2[sl] + c3[sl])
        fs[sl] = ssum.astype(jnp.float32) * (1.0 / QSCALE)
        fc[sl] = csum.astype(jnp.float32)

    pltpu.sync_copy(fs, out_hbm.at[0, wid])
    pltpu.sync_copy(fc, out_hbm.at[1, wid])


def _finish_kernel(p_ref, out_ref):
    s = jnp.sum(p_ref[0], axis=(0, 2))
    c = jnp.sum(p_ref[1], axis=(0, 2))
    nz = c > 0.0
    n = jnp.sum(jnp.where(nz, 1.0, 0.0))
    total = jnp.sum(jnp.where(nz, s / jnp.maximum(c, 1.0), 0.0))
    out_ref[0, 0] = total / jnp.maximum(n, 1.0)


def kernel(input, target):
    packed = pl.pallas_call(
        _pack_kernel,
        grid=(GRID_A,),
        in_specs=[
            pl.BlockSpec((BLK_A, COLS), lambda i: (i, 0)),
            pl.BlockSpec((BLK_A, COLS), lambda i: (i, 0)),
        ],
        out_specs=pl.BlockSpec((BLK_A, COLS), lambda i: (i, 0)),
        out_shape=jax.ShapeDtypeStruct((ROWS, COLS), jnp.int32),
        compiler_params=pltpu.CompilerParams(
            dimension_semantics=("arbitrary",),
        ),
    )(input, target)

    sc_hist = functools.partial(
        pl.kernel,
        mesh=plsc.VectorSubcoreMesh(core_axis_name="c", subcore_axis_name="s"),
        out_type=jax.ShapeDtypeStruct((2, NW, NSLOT * LANES), jnp.float32),
        scratch_types=[
            pltpu.VMEM((ROWS_CH, COLS), jnp.int32),
            pltpu.VMEM((ROWS_CH, COLS), jnp.int32),
        ] + [pltpu.VMEM((NSLOT * LANES,), jnp.int32) for _ in range(2 * NACC)] + [
            pltpu.VMEM((NSLOT * LANES,), jnp.float32),
            pltpu.VMEM((NSLOT * LANES,), jnp.float32),
            pltpu.SemaphoreType.DMA,
            pltpu.SemaphoreType.DMA,
        ],
        compiler_params=pltpu.CompilerParams(
            needs_layout_passes=False,
            skip_device_barrier=True,
        ),
    )(_sc_hist)
    partials = sc_hist(packed)

    partials = partials.reshape(2, NW, NSLOT, LANES)

    out = pl.pallas_call(
        _finish_kernel,
        out_specs=pl.BlockSpec(memory_space=pltpu.SMEM),
        out_shape=jax.ShapeDtypeStruct((1, 1), jnp.float32),
    )(partials)
    return out[0, 0]


# speedup vs baseline: 1.0908x; 1.0617x over previous
import functools

import jax
import jax.numpy as jnp
from jax import lax
from jax.experimental import pallas as pl
from jax.experimental.pallas import tpu as pltpu
from jax.experimental.pallas import tpu_sc as plsc

BINS = 30
ROWS, COLS = 16384, 256
TOT = ROWS * COLS

NC, NS, LANES = 2, 16, 16
NW = NC * NS

QSHIFT = 7
QSCALE = float(1 << QSHIFT)
QMAX = (1 << 11) - 1

BLK_A = 2048
GRID_A = ROWS // BLK_A

ROWS_W = ROWS // NW
ROWS_CH = 256
NCH = ROWS_W // ROWS_CH
PAIR_GROUPS = COLS // 32
NSLOT = 32


def _pack_kernel(x_ref, t_ref, out_ref):
    x = x_ref[...]
    t = t_ref[...]
    u = jnp.exp(-jnp.abs(x))
    one_u = 1.0 + u
    r = 1.0 / one_u
    sig = jnp.where(x >= 0.0, r, 1.0 - r)
    g30 = jnp.abs(sig - t) * float(BINS)
    b = jnp.minimum(jnp.floor(g30), float(BINS - 1)).astype(jnp.int32)
    e = jnp.maximum(x, 0.0) - x * t + jnp.log(one_u)
    q = jnp.minimum((e * QSCALE + 0.5).astype(jnp.int32), QMAX)
    w = jnp.bitwise_or(jnp.left_shift(q, 5), b)
    out_ref[...] = w.astype(jnp.uint16)


NACC = 4


def _sc_hist(packed_hbm, out_hbm, buf0, buf1,
             s0, s1, s2, s3, c0, c1, c2, c3, fs, fc, sem0, sem1):
    wid = lax.axis_index("c") * NS + lax.axis_index("s")
    row0 = wid * ROWS_W
    accs = (s0, s1, s2, s3)
    accc = (c0, c1, c2, c3)

    zeros16 = jnp.zeros((LANES,), jnp.int32)
    for j in range(NSLOT):
        for a in range(NACC):
            accs[a][pl.ds(j * LANES, LANES)] = zeros16
            accc[a][pl.ds(j * LANES, LANES)] = zeros16

    lane = lax.iota(jnp.int32, LANES)
    ones16 = jnp.ones((LANES,), jnp.int32)
    bufs = (buf0, buf1)
    sems = (sem0, sem1)

    def start(c):
        return pltpu.async_copy(
            packed_hbm.at[pl.ds(row0 + c * ROWS_CH, ROWS_CH), :],
            bufs[c % 2], sems[c % 2])

    def process(buf):
        @plsc.parallel_loop(0, ROWS_CH)
        def row_body(r):
            idxs, qs = [], []
            for k in range(PAIR_GROUPS):
                w32 = buf[r, pl.ds(k * 32, 32)]
                wi = plsc.bitcast(w32, jnp.int32)
                for v in (jnp.bitwise_and(wi, 0xFFFF),
                          lax.shift_right_logical(wi, 16)):
                    idxs.append(
                        jnp.bitwise_and(jnp.left_shift(v, 4), 0x1F0) + lane)
                    qs.append(lax.shift_right_logical(v, 5))
            for m in range(2 * PAIR_GROUPS):
                plsc.addupdate_scatter(accs[m % NACC], [idxs[m]], qs[m])
                plsc.addupdate_scatter(accc[m % NACC], [idxs[m]], ones16)

    descs = [start(0)]
    for c in range(NCH):
        if c + 1 < NCH:
            descs.append(start(c + 1))
        descs[c].wait()
        process(bufs[c % 2])

    for j in range(NSLOT):
        sl = pl.ds(j * LANES, LANES)
        ssum = (s0[sl] + s1[sl]) + (s2[sl] + s3[sl])
        csum = (c0[sl] + c1[sl]) + (c2[sl] + c3[sl])
        fs[sl] = ssum.astype(jnp.float32) * (1.0 / QSCALE)
        fc[sl] = csum.astype(jnp.float32)

    pltpu.sync_copy(fs, out_hbm.at[0, wid])
    pltpu.sync_copy(fc, out_hbm.at[1, wid])


def _finish_kernel(p_ref, out_ref):
    s = jnp.sum(p_ref[0], axis=(0, 2))
    c = jnp.sum(p_ref[1], axis=(0, 2))
    nz = c > 0.0
    n = jnp.sum(jnp.where(nz, 1.0, 0.0))
    total = jnp.sum(jnp.where(nz, s / jnp.maximum(c, 1.0), 0.0))
    out_ref[0, 0] = total / jnp.maximum(n, 1.0)


def kernel(input, target):
    packed = pl.pallas_call(
        _pack_kernel,
        grid=(GRID_A,),
        in_specs=[
            pl.BlockSpec((BLK_A, COLS), lambda i: (i, 0)),
            pl.BlockSpec((BLK_A, COLS), lambda i: (i, 0)),
        ],
        out_specs=pl.BlockSpec((BLK_A, COLS), lambda i: (i, 0)),
        out_shape=jax.ShapeDtypeStruct((ROWS, COLS), jnp.uint16),
        compiler_params=pltpu.CompilerParams(
            dimension_semantics=("arbitrary",),
        ),
    )(input, target)

    sc_hist = functools.partial(
        pl.kernel,
        mesh=plsc.VectorSubcoreMesh(core_axis_name="c", subcore_axis_name="s"),
        out_type=jax.ShapeDtypeStruct((2, NW, NSLOT * LANES), jnp.float32),
        scratch_types=[
            pltpu.VMEM((ROWS_CH, COLS), jnp.uint16),
            pltpu.VMEM((ROWS_CH, COLS), jnp.uint16),
        ] + [pltpu.VMEM((NSLOT * LANES,), jnp.int32) for _ in range(2 * NACC)] + [
            pltpu.VMEM((NSLOT * LANES,), jnp.float32),
            pltpu.VMEM((NSLOT * LANES,), jnp.float32),
            pltpu.SemaphoreType.DMA,
            pltpu.SemaphoreType.DMA,
        ],
        compiler_params=pltpu.CompilerParams(
            needs_layout_passes=False,
            skip_device_barrier=True,
        ),
    )(_sc_hist)
    partials = sc_hist(packed)

    partials = partials.reshape(2, NW, NSLOT, LANES)

    out = pl.pallas_call(
        _finish_kernel,
        out_specs=pl.BlockSpec(memory_space=pltpu.SMEM),
        out_shape=jax.ShapeDtypeStruct((1, 1), jnp.float32),
    )(partials)
    return out[0, 0]
